# baseline (speedup 1.0000x reference)
import functools
import math

import jax
import jax.numpy as jnp
from jax import lax
from jax.experimental import pallas as pl
from jax.experimental.pallas import tpu as pltpu
from jax.experimental.pallas import tpu_sc as plsc

L = 9
F = 4
B = 16384
N_SIDE = 2 ** (L - 1)
N_PIX = 12 * N_SIDE * N_SIDE

SUB = 8
COL = B // SUB
LK = L * 8

_OFFS = [(-1, -1), (-1, 0), (-1, 1), (0, -1), (0, 1), (1, -1), (1, 0), (1, 1)]

_ASIN_COEFFS = []
for _k in range(16):
    _ASIN_COEFFS.append(
        math.comb(2 * _k, _k) / (4.0 ** _k) / (2 * _k + 1)
    )


def _asin_small(t):
    t2 = t * t
    acc = jnp.full_like(t, _ASIN_COEFFS[-1])
    for c in reversed(_ASIN_COEFFS[:-1]):
        acc = acc * t2 + c
    return acc * t


def _dist_from_a(a):
    a = jnp.clip(a, 0.0, 1.0)
    sa = jnp.sqrt(a)
    sb = jnp.sqrt(1.0 - a)
    small = a <= 0.5
    t = jnp.where(small, sa, sb)
    p = _asin_small(t)
    half = jnp.where(small, p, (math.pi / 2.0) - p)
    return 2.0 * half


def _index_weight_body(lat_ref, lon_ref, idx_ref, w0_ref, w1_ref):
    lat = lat_ref[...]
    lon = lon_ref[...]

    per_level = []
    idx_raw = []
    valid_l = []
    dist_v = []
    max_idx = jnp.int32(-1)
    max_dv = jnp.float32(-1.0)

    for l in range(L):
        nside = 2 ** l
        n_lon = 4 * nside
        n_lat = 3 * nside
        i_lon = jnp.clip(jnp.floor(lon / 360.0 * n_lon), 0, n_lon - 1)
        i_lon = i_lon.astype(jnp.int32)
        i_lat = jnp.clip(jnp.floor((lat + 90.0) / 180.0 * n_lat), 0, n_lat - 1)
        i_lat = i_lat.astype(jnp.int32)
        th1 = (i_lat.astype(jnp.float32) + 0.5) / n_lat * math.pi
        ph1 = (i_lon.astype(jnp.float32) + 0.5) / n_lon * (2.0 * math.pi)
        cos_th1 = jnp.cos(th1)
        per_level.append((i_lat, i_lon, th1, ph1, cos_th1))

        for dlat, dlon in _OFFS:
            nlat = i_lat + dlat
            nlon = jnp.mod(i_lon + dlon, n_lon)
            valid = (nlat >= 0) & (nlat < n_lat)
            nidx = nlat * n_lon + nlon
            raw = jnp.where(valid, nidx, -1)
            idx_raw.append(raw)
            valid_l.append(valid)
            max_idx = jnp.maximum(max_idx, jnp.max(raw))

            th2 = (nlat.astype(jnp.float32) + 0.5) / n_lat * math.pi
            ph2 = (nlon.astype(jnp.float32) + 0.5) / n_lon * (2.0 * math.pi)
            dth = th2 - th1
            dph = ph2 - ph1
            sin_dth = jnp.sin(dth / 2.0)
            sin_dph = jnp.sin(dph / 2.0)
            a = sin_dth * sin_dth + cos_th1 * jnp.cos(th2) * sin_dph * sin_dph
            d = _dist_from_a(a)
            dist_v.append(d)
            max_dv = jnp.maximum(
                max_dv, jnp.max(jnp.where(valid, d, -1.0)))

    repl = max_idx + 1

    max_d = max_dv
    for l in range(L):
        nside = 2 ** l
        n_lon = 4 * nside
        n_lat = 3 * nside
        i_lat, _, th1, ph1, cos_th1 = per_level[l]
        i_lat2 = repl // n_lon
        i_lon2 = repl % n_lon
        th2 = (i_lat2.astype(jnp.float32) + 0.5) / n_lat * math.pi
        ph2 = (i_lon2.astype(jnp.float32) + 0.5) / n_lon * (2.0 * math.pi)
        th2v = jnp.broadcast_to(th2, th1.shape)
        ph2v = jnp.broadcast_to(ph2, ph1.shape)
        dth = th2v - th1
        dph = ph2v - ph1
        sin_dth = jnp.sin(dth / 2.0)
        sin_dph = jnp.sin(dph / 2.0)
        a = sin_dth * sin_dth + cos_th1 * jnp.cos(th2v) * sin_dph * sin_dph
        dm = _dist_from_a(a)
        border = (i_lat == 0) | (i_lat == n_lat - 1)
        max_d = jnp.maximum(max_d, jnp.max(jnp.where(border, dm, -1.0)))

    for l in range(L):
        for k in range(8):
            r = l * 8 + k
            valid = valid_l[r]
            gidx = l * N_PIX + jnp.where(valid, idx_raw[r], 0)
            idx_ref[r] = gidx >> 1
            wv = jnp.where(valid, max_d - dist_v[r], 0.0)
            odd = (gidx & 1) == 1
            w0_ref[r] = jnp.where(odd, 0.0, wv)
            w1_ref[r] = jnp.where(odd, wv, 0.0)


def _index_weight(lat, lon):
    return pl.pallas_call(
        _index_weight_body,
        out_shape=[
            jax.ShapeDtypeStruct((LK, SUB, COL), jnp.int32),
            jax.ShapeDtypeStruct((LK, SUB, COL), jnp.float32),
            jax.ShapeDtypeStruct((LK, SUB, COL), jnp.float32),
        ],
    )(lat, lon)


NROWS = LK * B
RW = 2 * F
IDX_MINOR = 128
IDX_ROWS = NROWS // IDX_MINOR
NWORKERS = 32
ROWS_PER_W = IDX_ROWS // NWORKERS
CH = 16


def _sc_gather_kernel(table_hbm, idx_hbm, out_hbm, idx_v, rows_v, sem):
    c = lax.axis_index("c")
    s = lax.axis_index("s")
    wid = s * 2 + c
    base = wid * ROWS_PER_W
    pltpu.sync_copy(idx_hbm.at[pl.ds(base, ROWS_PER_W)], idx_v)

    @pl.loop(0, ROWS_PER_W // CH)
    def _(i):
        cps = []
        for j in range(CH):
            cps.append(pltpu.async_copy(
                table_hbm.at[idx_v.at[i * CH + j]],
                rows_v.at[pl.ds(j * IDX_MINOR, IDX_MINOR)],
                sem,
            ))
        for cp in cps:
            cp.wait()
        pltpu.sync_copy(
            rows_v,
            out_hbm.at[pl.ds((base + i * CH) * IDX_MINOR, CH * IDX_MINOR)],
        )


def _sc_gather(table, idx2d):
    kern = pl.kernel(
        _sc_gather_kernel,
        out_type=jax.ShapeDtypeStruct((NROWS, RW), jnp.float32),
        mesh=plsc.VectorSubcoreMesh(core_axis_name="c", subcore_axis_name="s"),
        scratch_types=[
            pltpu.VMEM((ROWS_PER_W, IDX_MINOR), jnp.int32),
            pltpu.VMEM((CH * IDX_MINOR, RW), jnp.float32),
            pltpu.SemaphoreType.DMA,
        ],
        compiler_params=pltpu.CompilerParams(use_tc_tiling_on_sc=False),
    )
    return kern(table, idx2d)


WCOLS = RW * B
NCB = 8
CB = WCOLS // NCB


def _weighted_sum_body(g_ref, w_ref, o_ref):
    for l in range(L):
        gl = g_ref[l]
        wl = w_ref[l]
        o_ref[l] = jnp.sum(gl * wl, axis=0)
    zero = jnp.zeros((1, CB), jnp.float32)
    for r in range(L, 16):
        o_ref[r] = zero[0]


def _weighted_sum(g3, w3):
    return pl.pallas_call(
        _weighted_sum_body,
        grid=(NCB,),
        in_specs=[
            pl.BlockSpec((L, 8, CB), lambda i: (0, 0, i)),
            pl.BlockSpec((L, 8, CB), lambda i: (0, 0, i)),
        ],
        out_specs=pl.BlockSpec((16, CB), lambda i: (0, i)),
        out_shape=jax.ShapeDtypeStruct((16, WCOLS), jnp.float32),
    )(g3, w3)


def kernel(x, params):
    lat = x[:, 0].reshape(SUB, COL)
    lon = x[:, 1].reshape(SUB, COL)
    idx, w0, w1 = _index_weight(lat, lon)

    table = params.reshape(L * N_PIX // 2, RW)
    idx2d = idx.reshape(IDX_ROWS, IDX_MINOR)
    g = _sc_gather(table, idx2d)

    w8 = jnp.repeat(
        jnp.stack([w0.reshape(LK, B), w1.reshape(LK, B)], axis=-1),
        F, axis=2,
    ).reshape(L, 8, WCOLS)
    g3 = g.reshape(L, 8, WCOLS)
    out8 = _weighted_sum(g3, w8)[:L]

    out = (
        out8.reshape(L, B, 2, F).sum(axis=2)
        .transpose(1, 2, 0).reshape(B, L * F)
    )
    return out
